# baseline (speedup 1.0000x reference)
import functools

import jax
import jax.numpy as jnp
from jax import lax
from jax.experimental import pallas as pl
from jax.experimental.pallas import tpu as pltpu
from jax.experimental.pallas import tpu_sc as plsc

N_ENT = 100000
N_REL = 1000
N_TRIPLES = 2000000
B = 4096

NTILES = 32
T_PER = N_TRIPLES // NTILES
CHUNK = 8192
NCHUNK = 8
LANES = 16
UNROLL = 8
PAIR_LO = 0

FILT_LOG2 = 21
FILT_WORDS = (1 << FILT_LOG2) // 32
HASH_A = -1640531535
HASH_C = -2049600905


def _sc_body(d_hbm, qhi_hbm, qlo_hbm, filt_hbm, out_hbm,
             qhi_v, qlo_v, filt_v, match_v, dh_c, dr_c, dt_c):
    i32 = jnp.int32
    wid = lax.axis_index("s") * i32(2) + lax.axis_index("c")
    base = wid * i32(T_PER)
    lane2 = jnp.arange(LANES, dtype=jnp.int32) * i32(2) + i32(PAIR_LO)

    pltpu.sync_copy(qhi_hbm, qhi_v)
    pltpu.sync_copy(qlo_hbm, qlo_v)
    pltpu.sync_copy(filt_hbm, filt_v)

    def _zero(i, carry):
        match_v[pl.ds(i * i32(LANES), LANES)] = jnp.zeros((LANES,), jnp.int32)
        return carry
    lax.fori_loop(i32(0), i32(B // LANES), _zero, i32(0))

    def _chunk(c, carry):
        start = base + jnp.minimum(c * i32(CHUNK), i32(T_PER - CHUNK))
        ws = start + start
        row = i32(2 * N_TRIPLES)
        pltpu.sync_copy(d_hbm.at[pl.ds(pl.multiple_of(ws, 8), 2 * CHUNK)],
                        dh_c)
        pltpu.sync_copy(d_hbm.at[pl.ds(pl.multiple_of(ws + row, 8), 2 * CHUNK)],
                        dr_c)
        pltpu.sync_copy(
            d_hbm.at[pl.ds(pl.multiple_of(ws + row + row, 8), 2 * CHUNK)],
            dt_c)

        def _vec(i, inner):
            hs, ls, hit = [], [], None
            for u in range(UNROLL):
                off = (i * i32(UNROLL) + i32(u)) * i32(LANES)
                idx2 = off + off + lane2
                h = plsc.load_gather(dh_c, [idx2])
                r = plsc.load_gather(dr_c, [idx2])
                t = plsc.load_gather(dt_c, [idx2])
                l = r * i32(N_ENT) + t
                hs.append(h)
                ls.append(l)
                hv = h * i32(HASH_A) + l * i32(HASH_C)
                idx = (hv >> 11) & i32((1 << FILT_LOG2) - 1)
                w = plsc.load_gather(filt_v, [idx >> 5])
                b = (w >> (idx & i32(31))) & i32(1)
                hit = b if hit is None else (hit | b)

            @pl.when(jnp.max(hit) > i32(0))
            def _search():
                ks = [jnp.zeros((LANES,), jnp.int32) for _ in range(UNROLL)]
                for bit in (2048, 1024, 512, 256, 128, 64, 32, 16, 8, 4, 2, 1):
                    for u in range(UNROLL):
                        cand = ks[u] + i32(bit)
                        qh = plsc.load_gather(qhi_v, [cand - i32(1)])
                        ql = plsc.load_gather(qlo_v, [cand - i32(1)])
                        less = (qh < hs[u]) | ((qh == hs[u]) & (ql < ls[u]))
                        ks[u] = jnp.where(less, cand, ks[u])
                for u in range(UNROLL):
                    pos = jnp.minimum(ks[u], i32(B - 1))
                    qh = plsc.load_gather(qhi_v, [pos])
                    ql = plsc.load_gather(qlo_v, [pos])
                    m = (qh == hs[u]) & (ql == ls[u])
                    plsc.store_scatter(match_v, [pos],
                                       jnp.ones((LANES,), jnp.int32), mask=m)
            return inner
        lax.fori_loop(i32(0), i32(CHUNK // (LANES * UNROLL)), _vec, i32(0))
        return carry
    lax.fori_loop(i32(0), i32(NCHUNK), _chunk, i32(0))

    pltpu.sync_copy(match_v, out_hbm.at[wid])


def _sc_match(d32, qhi_s, qlo_s, filt):
    mesh = plsc.VectorSubcoreMesh(core_axis_name="c", subcore_axis_name="s")
    fn = functools.partial(
        pl.kernel, _sc_body, mesh=mesh,
        compiler_params=pltpu.CompilerParams(needs_layout_passes=False),
        out_type=jax.ShapeDtypeStruct((NTILES, B), jnp.int32),
        scratch_types=[
            pltpu.VMEM((B,), jnp.int32),
            pltpu.VMEM((B,), jnp.int32),
            pltpu.VMEM((FILT_WORDS,), jnp.int32),
            pltpu.VMEM((B,), jnp.int32),
            pltpu.VMEM((2 * CHUNK,), jnp.int32),
            pltpu.VMEM((2 * CHUNK,), jnp.int32),
            pltpu.VMEM((2 * CHUNK,), jnp.int32),
        ],
    )()
    return fn(d32, qhi_s, qlo_s, filt)


_TC_BLK = 256


def _tc_body(p_ref, qhis_ref, qlos_ref, qhi_ref, qlo_ref, o_ref):
    mvec = jnp.max(p_ref[...], axis=0)
    hs = qhis_ref[...]
    ls = qlos_ref[...]
    for k in range(B // _TC_BLK):
        qh = qhi_ref[pl.ds(k * _TC_BLK, _TC_BLK)]
        ql = qlo_ref[pl.ds(k * _TC_BLK, _TC_BLK)]
        eq = (qh[:, None] == hs[None, :]) & (ql[:, None] == ls[None, :])
        val = jnp.where(eq, mvec[None, :], jnp.int32(0))
        found = jnp.max(val, axis=1) > 0
        o_ref[pl.ds(k * _TC_BLK, _TC_BLK)] = jnp.where(
            found, jnp.float32(5.0), jnp.float32(-5.0))


def _tc_resolve(partial, qhi_s, qlo_s, qhi, qlo):
    return pl.pallas_call(
        _tc_body,
        out_shape=jax.ShapeDtypeStruct((B,), jnp.float32),
    )(partial, qhi_s, qlo_s, qhi, qlo)


def kernel(heads, rels, tails, data):
    d32 = lax.bitcast_convert_type(data, jnp.int32).reshape(6 * N_TRIPLES)

    qlo64 = rels * N_ENT + tails
    skey = heads * jnp.int64(1 << 27) + qlo64
    skey_s = jnp.sort(skey)
    qhi_s = (skey_s >> 27).astype(jnp.int32)
    qlo_s = (skey_s & ((1 << 27) - 1)).astype(jnp.int32)
    qhi = heads.astype(jnp.int32)
    qlo = qlo64.astype(jnp.int32)

    hv = qhi * jnp.int32(HASH_A) + qlo * jnp.int32(HASH_C)
    idx = (hv >> 11) & jnp.int32((1 << FILT_LOG2) - 1)
    bits = jnp.zeros((1 << FILT_LOG2,), jnp.bool_).at[idx].set(True)
    weights = jnp.left_shift(jnp.uint32(1), jnp.arange(32, dtype=jnp.uint32))
    filt_u = (bits.reshape(FILT_WORDS, 32).astype(jnp.uint32) * weights
              ).sum(axis=1, dtype=jnp.uint32)
    filt = lax.bitcast_convert_type(filt_u, jnp.int32)

    partial = _sc_match(d32, qhi_s, qlo_s, filt)
    return _tc_resolve(partial, qhi_s, qlo_s, qhi, qlo)


# speedup vs baseline: 8.2074x; 8.2074x over previous
import functools

import jax
import jax.numpy as jnp
from jax import lax
from jax.experimental import pallas as pl
from jax.experimental.pallas import tpu as pltpu
from jax.experimental.pallas import tpu_sc as plsc

N_ENT = 100000
N_REL = 1000
N_TRIPLES = 2000000
B = 4096

NTILES = 32
T_PER = N_TRIPLES // NTILES
CHUNK = 8192
NCHUNK = 8
LANES = 16
UNROLL = 8

FILT_LOG2 = 21
FILT_WORDS = (1 << FILT_LOG2) // 32
HASH_A = -1640531535
HASH_C = -2049600905


def _sc_body(d_hbm, qhi_hbm, qlo_hbm, filt_hbm, out_hbm,
             qhi_v, qlo_v, filt_v, match_v, dh_c, dr_c, dt_c):
    i32 = jnp.int32
    wid = lax.axis_index("s") * i32(2) + lax.axis_index("c")
    base = (wid * i32(T_PER)) & i32(~7)
    nxt = ((wid + i32(1)) * i32(T_PER)) & i32(~7)

    pltpu.sync_copy(qhi_hbm, qhi_v)
    pltpu.sync_copy(qlo_hbm, qlo_v)
    pltpu.sync_copy(filt_hbm, filt_v)

    def _zero(i, carry):
        match_v[pl.ds(i * i32(LANES), LANES)] = jnp.zeros((LANES,), jnp.int32)
        return carry
    lax.fori_loop(i32(0), i32(B // LANES), _zero, i32(0))

    def _chunk(c, carry):
        ws = jnp.minimum(base + c * i32(CHUNK), nxt - i32(CHUNK))
        row = i32(N_TRIPLES)
        pltpu.sync_copy(d_hbm.at[pl.ds(pl.multiple_of(ws, 8), CHUNK)],
                        dh_c)
        pltpu.sync_copy(d_hbm.at[pl.ds(pl.multiple_of(ws + row, 8), CHUNK)],
                        dr_c)
        pltpu.sync_copy(
            d_hbm.at[pl.ds(pl.multiple_of(ws + row + row, 8), CHUNK)],
            dt_c)

        def _vec(i, inner):
            hs, ls, hit = [], [], None
            for u in range(UNROLL):
                off = (i * i32(UNROLL) + i32(u)) * i32(LANES)
                h = dh_c[pl.ds(off, LANES)]
                r = dr_c[pl.ds(off, LANES)]
                t = dt_c[pl.ds(off, LANES)]
                l = r * i32(N_ENT) + t
                hs.append(h)
                ls.append(l)
                hv = h * i32(HASH_A) + l * i32(HASH_C)
                idx = (hv >> 11) & i32((1 << FILT_LOG2) - 1)
                w = plsc.load_gather(filt_v, [idx >> 5])
                b = (w >> (idx & i32(31))) & i32(1)
                hit = b if hit is None else (hit | b)

            @pl.when(jnp.max(hit) > i32(0))
            def _search():
                ks = [jnp.zeros((LANES,), jnp.int32) for _ in range(UNROLL)]
                for bit in (2048, 1024, 512, 256, 128, 64, 32, 16, 8, 4, 2, 1):
                    for u in range(UNROLL):
                        cand = ks[u] + i32(bit)
                        qh = plsc.load_gather(qhi_v, [cand - i32(1)])
                        ql = plsc.load_gather(qlo_v, [cand - i32(1)])
                        less = (qh < hs[u]) | ((qh == hs[u]) & (ql < ls[u]))
                        ks[u] = jnp.where(less, cand, ks[u])
                for u in range(UNROLL):
                    pos = jnp.minimum(ks[u], i32(B - 1))
                    qh = plsc.load_gather(qhi_v, [pos])
                    ql = plsc.load_gather(qlo_v, [pos])
                    m = (qh == hs[u]) & (ql == ls[u])
                    plsc.store_scatter(match_v, [pos],
                                       jnp.ones((LANES,), jnp.int32), mask=m)
            return inner
        lax.fori_loop(i32(0), i32(CHUNK // (LANES * UNROLL)), _vec, i32(0))
        return carry
    lax.fori_loop(i32(0), i32(NCHUNK), _chunk, i32(0))

    pltpu.sync_copy(match_v, out_hbm.at[wid])


def _sc_match(d32, qhi_s, qlo_s, filt):
    mesh = plsc.VectorSubcoreMesh(core_axis_name="c", subcore_axis_name="s")
    fn = functools.partial(
        pl.kernel, _sc_body, mesh=mesh,
        compiler_params=pltpu.CompilerParams(needs_layout_passes=False),
        out_type=jax.ShapeDtypeStruct((NTILES, B), jnp.int32),
        scratch_types=[
            pltpu.VMEM((B,), jnp.int32),
            pltpu.VMEM((B,), jnp.int32),
            pltpu.VMEM((FILT_WORDS,), jnp.int32),
            pltpu.VMEM((B,), jnp.int32),
            pltpu.VMEM((CHUNK,), jnp.int32),
            pltpu.VMEM((CHUNK,), jnp.int32),
            pltpu.VMEM((CHUNK,), jnp.int32),
        ],
    )()
    return fn(d32, qhi_s, qlo_s, filt)


_TC_BLK = 256


def _tc_body(p_ref, qhis_ref, qlos_ref, qhi_ref, qlo_ref, o_ref):
    mvec = jnp.max(p_ref[...], axis=0)
    hs = qhis_ref[...]
    ls = qlos_ref[...]
    for k in range(B // _TC_BLK):
        qh = qhi_ref[pl.ds(k * _TC_BLK, _TC_BLK)]
        ql = qlo_ref[pl.ds(k * _TC_BLK, _TC_BLK)]
        eq = (qh[:, None] == hs[None, :]) & (ql[:, None] == ls[None, :])
        val = jnp.where(eq, mvec[None, :], jnp.int32(0))
        found = jnp.max(val, axis=1) > 0
        o_ref[pl.ds(k * _TC_BLK, _TC_BLK)] = jnp.where(
            found, jnp.float32(5.0), jnp.float32(-5.0))


def _tc_resolve(partial, qhi_s, qlo_s, qhi, qlo):
    return pl.pallas_call(
        _tc_body,
        out_shape=jax.ShapeDtypeStruct((B,), jnp.float32),
    )(partial, qhi_s, qlo_s, qhi, qlo)


def kernel(heads, rels, tails, data):
    d32 = data.astype(jnp.int32).reshape(3 * N_TRIPLES)

    qlo64 = rels * N_ENT + tails
    skey = heads * jnp.int64(1 << 27) + qlo64
    skey_s = jnp.sort(skey)
    qhi_s = (skey_s >> 27).astype(jnp.int32)
    qlo_s = (skey_s & ((1 << 27) - 1)).astype(jnp.int32)
    qhi = heads.astype(jnp.int32)
    qlo = qlo64.astype(jnp.int32)

    hv = qhi * jnp.int32(HASH_A) + qlo * jnp.int32(HASH_C)
    idx = (hv >> 11) & jnp.int32((1 << FILT_LOG2) - 1)
    bits = jnp.zeros((1 << FILT_LOG2,), jnp.bool_).at[idx].set(True)
    weights = jnp.left_shift(jnp.uint32(1), jnp.arange(32, dtype=jnp.uint32))
    filt_u = (bits.reshape(FILT_WORDS, 32).astype(jnp.uint32) * weights
              ).sum(axis=1, dtype=jnp.uint32)
    filt = lax.bitcast_convert_type(filt_u, jnp.int32)

    partial = _sc_match(d32, qhi_s, qlo_s, filt)
    return _tc_resolve(partial, qhi_s, qlo_s, qhi, qlo)


# speedup vs baseline: 12.7043x; 1.5479x over previous
import functools

import jax
import jax.numpy as jnp
from jax import lax
from jax.experimental import pallas as pl
from jax.experimental.pallas import tpu as pltpu
from jax.experimental.pallas import tpu_sc as plsc

N_ENT = 100000
N_REL = 1000
N_TRIPLES = 2000000
B = 4096

NTILES = 32
T_PER = N_TRIPLES // NTILES
CHUNK = 8192
NCHUNK = 8
LANES = 16
UNROLL = 8

FILT_LOG2 = 21
FILT_WORDS = (1 << FILT_LOG2) // 32
HASH_A = -1640531535
HASH_C = -2049600905


def _sc_body(dh_hbm, dr_hbm, dt_hbm, qhi_hbm, qlo_hbm, filt_hbm, out_hbm,
             qhi_v, qlo_v, filt_v, match_v, dh_c, dr_c, dt_c):
    i32 = jnp.int32
    wid = lax.axis_index("s") * i32(2) + lax.axis_index("c")
    base = (wid * i32(T_PER)) & i32(~7)
    nxt = ((wid + i32(1)) * i32(T_PER)) & i32(~7)

    pltpu.sync_copy(qhi_hbm, qhi_v)
    pltpu.sync_copy(qlo_hbm, qlo_v)
    pltpu.sync_copy(filt_hbm, filt_v)

    def _zero(i, carry):
        match_v[pl.ds(i * i32(LANES), LANES)] = jnp.zeros((LANES,), jnp.int32)
        return carry
    lax.fori_loop(i32(0), i32(B // LANES), _zero, i32(0))

    def _chunk(c, carry):
        ws = pl.multiple_of(
            jnp.minimum(base + c * i32(CHUNK), nxt - i32(CHUNK)), 8)
        pltpu.sync_copy(dh_hbm.at[pl.ds(ws, CHUNK)], dh_c)
        pltpu.sync_copy(dr_hbm.at[pl.ds(ws, CHUNK)], dr_c)
        pltpu.sync_copy(dt_hbm.at[pl.ds(ws, CHUNK)], dt_c)

        def _vec(i, inner):
            hs, ls, hit = [], [], None
            for u in range(UNROLL):
                off = (i * i32(UNROLL) + i32(u)) * i32(LANES)
                h = dh_c[pl.ds(off, LANES)]
                r = dr_c[pl.ds(off, LANES)]
                t = dt_c[pl.ds(off, LANES)]
                l = r * i32(N_ENT) + t
                hs.append(h)
                ls.append(l)
                hv = h * i32(HASH_A) + l * i32(HASH_C)
                idx = (hv >> 11) & i32((1 << FILT_LOG2) - 1)
                w = plsc.load_gather(filt_v, [idx >> 5])
                b = (w >> (idx & i32(31))) & i32(1)
                hit = b if hit is None else (hit | b)

            @pl.when(jnp.max(hit) > i32(0))
            def _search():
                ks = [jnp.zeros((LANES,), jnp.int32) for _ in range(UNROLL)]
                for bit in (2048, 1024, 512, 256, 128, 64, 32, 16, 8, 4, 2, 1):
                    for u in range(UNROLL):
                        cand = ks[u] + i32(bit)
                        qh = plsc.load_gather(qhi_v, [cand - i32(1)])
                        ql = plsc.load_gather(qlo_v, [cand - i32(1)])
                        less = (qh < hs[u]) | ((qh == hs[u]) & (ql < ls[u]))
                        ks[u] = jnp.where(less, cand, ks[u])
                for u in range(UNROLL):
                    pos = jnp.minimum(ks[u], i32(B - 1))
                    qh = plsc.load_gather(qhi_v, [pos])
                    ql = plsc.load_gather(qlo_v, [pos])
                    m = (qh == hs[u]) & (ql == ls[u])
                    plsc.store_scatter(match_v, [pos],
                                       jnp.ones((LANES,), jnp.int32), mask=m)
            return inner
        lax.fori_loop(i32(0), i32(CHUNK // (LANES * UNROLL)), _vec, i32(0))
        return carry
    lax.fori_loop(i32(0), i32(NCHUNK), _chunk, i32(0))

    pltpu.sync_copy(match_v, out_hbm.at[wid])


def _sc_match(dh, dr, dt, qhi_s, qlo_s, filt):
    mesh = plsc.VectorSubcoreMesh(core_axis_name="c", subcore_axis_name="s")
    fn = functools.partial(
        pl.kernel, _sc_body, mesh=mesh,
        compiler_params=pltpu.CompilerParams(needs_layout_passes=False),
        out_type=jax.ShapeDtypeStruct((NTILES, B), jnp.int32),
        scratch_types=[
            pltpu.VMEM((B,), jnp.int32),
            pltpu.VMEM((B,), jnp.int32),
            pltpu.VMEM((FILT_WORDS,), jnp.int32),
            pltpu.VMEM((B,), jnp.int32),
            pltpu.VMEM((CHUNK,), jnp.int32),
            pltpu.VMEM((CHUNK,), jnp.int32),
            pltpu.VMEM((CHUNK,), jnp.int32),
        ],
    )()
    return fn(dh, dr, dt, qhi_s, qlo_s, filt)


_TC_BLK = 256


def _tc_body(p_ref, qhis_ref, qlos_ref, qhi_ref, qlo_ref, o_ref):
    mvec = jnp.max(p_ref[...], axis=0)
    hs = qhis_ref[...]
    ls = qlos_ref[...]
    for k in range(B // _TC_BLK):
        qh = qhi_ref[pl.ds(k * _TC_BLK, _TC_BLK)]
        ql = qlo_ref[pl.ds(k * _TC_BLK, _TC_BLK)]
        eq = (qh[:, None] == hs[None, :]) & (ql[:, None] == ls[None, :])
        val = jnp.where(eq, mvec[None, :], jnp.int32(0))
        found = jnp.max(val, axis=1) > 0
        o_ref[pl.ds(k * _TC_BLK, _TC_BLK)] = jnp.where(
            found, jnp.float32(5.0), jnp.float32(-5.0))


def _tc_resolve(partial, qhi_s, qlo_s, qhi, qlo):
    return pl.pallas_call(
        _tc_body,
        out_shape=jax.ShapeDtypeStruct((B,), jnp.float32),
    )(partial, qhi_s, qlo_s, qhi, qlo)


def kernel(heads, rels, tails, data):
    dh = data[0].astype(jnp.int32)
    dr = data[1].astype(jnp.int32)
    dt = data[2].astype(jnp.int32)

    qlo64 = rels * N_ENT + tails
    skey = heads * jnp.int64(1 << 27) + qlo64
    skey_s = jnp.sort(skey)
    qhi_s = (skey_s >> 27).astype(jnp.int32)
    qlo_s = (skey_s & ((1 << 27) - 1)).astype(jnp.int32)
    qhi = heads.astype(jnp.int32)
    qlo = qlo64.astype(jnp.int32)

    hv = qhi * jnp.int32(HASH_A) + qlo * jnp.int32(HASH_C)
    idx = (hv >> 11) & jnp.int32((1 << FILT_LOG2) - 1)
    bits = jnp.zeros((1 << FILT_LOG2,), jnp.bool_).at[idx].set(True)
    weights = jnp.left_shift(jnp.uint32(1), jnp.arange(32, dtype=jnp.uint32))
    filt_u = (bits.reshape(FILT_WORDS, 32).astype(jnp.uint32) * weights
              ).sum(axis=1, dtype=jnp.uint32)
    filt = lax.bitcast_convert_type(filt_u, jnp.int32)

    partial = _sc_match(dh, dr, dt, qhi_s, qlo_s, filt)
    return _tc_resolve(partial, qhi_s, qlo_s, qhi, qlo)


# speedup vs baseline: 13.1938x; 1.0385x over previous
import functools

import jax
import jax.numpy as jnp
from jax import lax
from jax.experimental import pallas as pl
from jax.experimental.pallas import tpu as pltpu
from jax.experimental.pallas import tpu_sc as plsc

N_ENT = 100000
N_REL = 1000
N_TRIPLES = 2000000
B = 4096

NTILES = 32
T_PER = N_TRIPLES // NTILES
CHUNK = 4096
NCHUNK = 16
LANES = 16
UNROLL = 8

FILT_LOG2 = 21
FILT_WORDS = (1 << FILT_LOG2) // 32
HASH_A = -1640531535
HASH_C = -2049600905


def _sc_body(dh_hbm, dr_hbm, dt_hbm, qhi_hbm, qlo_hbm, filt_hbm, out_hbm,
             qhi_v, qlo_v, filt_v, match_v,
             dh_a, dr_a, dt_a, dh_b, dr_b, dt_b, sem_a, sem_b):
    i32 = jnp.int32
    wid = lax.axis_index("s") * i32(2) + lax.axis_index("c")
    base = (wid * i32(T_PER)) & i32(~7)
    nxt = ((wid + i32(1)) * i32(T_PER)) & i32(~7)

    def _fire(cidx, bufs, sem):
        ws = pl.multiple_of(
            jnp.minimum(base + cidx * i32(CHUNK), nxt - i32(CHUNK)), 8)
        pltpu.async_copy(dh_hbm.at[pl.ds(ws, CHUNK)], bufs[0], sem)
        pltpu.async_copy(dr_hbm.at[pl.ds(ws, CHUNK)], bufs[1], sem)
        pltpu.async_copy(dt_hbm.at[pl.ds(ws, CHUNK)], bufs[2], sem)

    def _drain(bufs, sem):
        for bf in bufs:
            pltpu.make_async_copy(dh_hbm.at[pl.ds(i32(0), CHUNK)], bf,
                                  sem).wait()

    pltpu.sync_copy(qhi_hbm, qhi_v)
    pltpu.sync_copy(qlo_hbm, qlo_v)
    pltpu.sync_copy(filt_hbm, filt_v)

    def _zero(i, carry):
        match_v[pl.ds(i * i32(LANES), LANES)] = jnp.zeros((LANES,), jnp.int32)
        return carry
    lax.fori_loop(i32(0), i32(B // LANES), _zero, i32(0))

    def _compute(dh_c, dr_c, dt_c):
        def _vec(i, inner):
            hs, ls, hit = [], [], None
            for u in range(UNROLL):
                off = (i * i32(UNROLL) + i32(u)) * i32(LANES)
                h = dh_c[pl.ds(off, LANES)]
                r = dr_c[pl.ds(off, LANES)]
                t = dt_c[pl.ds(off, LANES)]
                l = r * i32(N_ENT) + t
                hs.append(h)
                ls.append(l)
                hv = h * i32(HASH_A) + l * i32(HASH_C)
                idx = (hv >> 11) & i32((1 << FILT_LOG2) - 1)
                w = plsc.load_gather(filt_v, [idx >> 5])
                b = (w >> (idx & i32(31))) & i32(1)
                hit = b if hit is None else (hit | b)

            @pl.when(jnp.max(hit) > i32(0))
            def _search():
                ks = [jnp.zeros((LANES,), jnp.int32) for _ in range(UNROLL)]
                for bit in (2048, 1024, 512, 256, 128, 64, 32, 16, 8, 4, 2, 1):
                    for u in range(UNROLL):
                        cand = ks[u] + i32(bit)
                        qh = plsc.load_gather(qhi_v, [cand - i32(1)])
                        ql = plsc.load_gather(qlo_v, [cand - i32(1)])
                        less = (qh < hs[u]) | ((qh == hs[u]) & (ql < ls[u]))
                        ks[u] = jnp.where(less, cand, ks[u])
                for u in range(UNROLL):
                    pos = jnp.minimum(ks[u], i32(B - 1))
                    qh = plsc.load_gather(qhi_v, [pos])
                    ql = plsc.load_gather(qlo_v, [pos])
                    m = (qh == hs[u]) & (ql == ls[u])
                    plsc.store_scatter(match_v, [pos],
                                       jnp.ones((LANES,), jnp.int32), mask=m)
            return inner
        lax.fori_loop(i32(0), i32(CHUNK // (LANES * UNROLL)), _vec, i32(0))

    bufs_a = (dh_a, dr_a, dt_a)
    bufs_b = (dh_b, dr_b, dt_b)
    _fire(i32(0), bufs_a, sem_a)

    def _pair(i, carry):
        c0 = i + i
        _drain(bufs_a, sem_a)
        _fire(c0 + i32(1), bufs_b, sem_b)
        _compute(*bufs_a)
        _drain(bufs_b, sem_b)
        _fire(jnp.minimum(c0 + i32(2), i32(NCHUNK - 1)), bufs_a, sem_a)
        _compute(*bufs_b)
        return carry
    lax.fori_loop(i32(0), i32(NCHUNK // 2), _pair, i32(0))
    _drain(bufs_a, sem_a)

    pltpu.sync_copy(match_v, out_hbm.at[wid])


def _sc_match(dh, dr, dt, qhi_s, qlo_s, filt):
    mesh = plsc.VectorSubcoreMesh(core_axis_name="c", subcore_axis_name="s")
    fn = functools.partial(
        pl.kernel, _sc_body, mesh=mesh,
        compiler_params=pltpu.CompilerParams(needs_layout_passes=False),
        out_type=jax.ShapeDtypeStruct((NTILES, B), jnp.int32),
        scratch_types=[
            pltpu.VMEM((B,), jnp.int32),
            pltpu.VMEM((B,), jnp.int32),
            pltpu.VMEM((FILT_WORDS,), jnp.int32),
            pltpu.VMEM((B,), jnp.int32),
            pltpu.VMEM((CHUNK,), jnp.int32),
            pltpu.VMEM((CHUNK,), jnp.int32),
            pltpu.VMEM((CHUNK,), jnp.int32),
            pltpu.VMEM((CHUNK,), jnp.int32),
            pltpu.VMEM((CHUNK,), jnp.int32),
            pltpu.VMEM((CHUNK,), jnp.int32),
            pltpu.SemaphoreType.DMA,
            pltpu.SemaphoreType.DMA,
        ],
    )()
    return fn(dh, dr, dt, qhi_s, qlo_s, filt)


_TC_BLK = 256


def _tc_body(p_ref, qhis_ref, qlos_ref, qhi_ref, qlo_ref, o_ref):
    mvec = jnp.max(p_ref[...], axis=0)
    hs = qhis_ref[...]
    ls = qlos_ref[...]
    for k in range(B // _TC_BLK):
        qh = qhi_ref[pl.ds(k * _TC_BLK, _TC_BLK)]
        ql = qlo_ref[pl.ds(k * _TC_BLK, _TC_BLK)]
        eq = (qh[:, None] == hs[None, :]) & (ql[:, None] == ls[None, :])
        val = jnp.where(eq, mvec[None, :], jnp.int32(0))
        found = jnp.max(val, axis=1) > 0
        o_ref[pl.ds(k * _TC_BLK, _TC_BLK)] = jnp.where(
            found, jnp.float32(5.0), jnp.float32(-5.0))


def _tc_resolve(partial, qhi_s, qlo_s, qhi, qlo):
    return pl.pallas_call(
        _tc_body,
        out_shape=jax.ShapeDtypeStruct((B,), jnp.float32),
    )(partial, qhi_s, qlo_s, qhi, qlo)


def kernel(heads, rels, tails, data):
    dh = data[0].astype(jnp.int32)
    dr = data[1].astype(jnp.int32)
    dt = data[2].astype(jnp.int32)

    qhi = heads.astype(jnp.int32)
    qlo = (rels * N_ENT + tails).astype(jnp.int32)
    qhi_s, qlo_s = lax.sort((qhi, qlo), num_keys=2)

    hv = qhi * jnp.int32(HASH_A) + qlo * jnp.int32(HASH_C)
    idx = (hv >> 11) & jnp.int32((1 << FILT_LOG2) - 1)
    bits = jnp.zeros((1 << FILT_LOG2,), jnp.bool_).at[idx].set(True)
    weights = jnp.left_shift(jnp.uint32(1), jnp.arange(32, dtype=jnp.uint32))
    filt_u = (bits.reshape(FILT_WORDS, 32).astype(jnp.uint32) * weights
              ).sum(axis=1, dtype=jnp.uint32)
    filt = lax.bitcast_convert_type(filt_u, jnp.int32)

    partial = _sc_match(dh, dr, dt, qhi_s, qlo_s, filt)
    return _tc_resolve(partial, qhi_s, qlo_s, qhi, qlo)
